# baseline (speedup 1.0000x reference)
import jax
import jax.numpy as jnp
from jax import lax
from jax.experimental import pallas as pl
from jax.experimental.pallas import tpu as pltpu
from jax.experimental.pallas import tpu_sc as plsc

F32 = jnp.float32
H = 128
E = 160000
L = 20000
LP = 20480

_NT = ["patient", "symptom", "procedure", "disease", "drug"]
_N = {"patient": 10000, "symptom": 5000, "procedure": 5000, "disease": 3000,
      "drug": 2000}
_PAD = {"patient": 10240, "symptom": 5120, "procedure": 5120, "disease": 3072,
        "drug": 2048}
_ET = [("patient", "symptom"), ("symptom", "patient"),
       ("patient", "procedure"), ("procedure", "patient"),
       ("patient", "disease"), ("disease", "patient"),
       ("patient", "drug"), ("drug", "patient")]

_ACC_ROWS = 10240


def _make_agg(items, with_counts):
  n_tables = 1 + max(it[1] for it in items)
  n_pairs = 1 + max(it[2] for it in items)
  n_outs = 1 + max(it[6] for it in items)
  n_cnts = (1 + max(it[7] for it in items)) if with_counts else 0
  max_e = 64 * max(it[4] for it in items)

  def body(*refs):
    k = 0
    tables = refs[k:k + n_tables]; k += n_tables
    pairs = [(refs[k + 2 * i], refs[k + 2 * i + 1]) for i in range(n_pairs)]
    k += 2 * n_pairs
    outs = refs[k:k + n_outs]; k += n_outs
    cnts = refs[k:k + n_cnts]; k += n_cnts
    sidx, didx, dst_st, rows, acc, sem, ssem = refs[k:k + 7]
    k += 7
    cntv = refs[k] if with_counts else None

    cid = lax.axis_index("c")
    sid = lax.axis_index("s")

    for (core, ti, pi, row_base, nch, n_pad, oi, ci) in items:

      @pl.when(cid == core)
      def _item(table=tables[ti], src=pairs[pi][0], dst=pairs[pi][1],
                aout=outs[oi], ci=ci, row_base=row_base, nch=nch,
                n_pad=n_pad):
        rpt = n_pad // 16
        base = sid * rpt
        epp = 64 * nch

        ebase = (row_base + sid) * epp
        pltpu.sync_copy(src.at[pl.ds(ebase, epp)], sidx.at[pl.ds(0, epp)])
        pltpu.sync_copy(dst.at[pl.ds(ebase, epp)], didx.at[pl.ds(0, epp)])

        def _zb(i, _):
          rows[0, i // 8, pl.ds((i % 8) * 16, 16)] = jnp.zeros((16,), F32)
          return 0
        lax.fori_loop(0, 512, _zb, 0)

        def _zr(j, _):
          pltpu.async_copy(rows.at[0], acc.at[pl.ds(base + j * 64, 64)], sem)
          return 0
        lax.fori_loop(0, rpt // 64, _zr, 0)

        def _zw(j, _):
          pltpu.make_async_copy(rows.at[0], acc.at[pl.ds(base, 64)],
                                sem).wait()
          return 0
        lax.fori_loop(0, rpt // 64, _zw, 0)

        if with_counts:
          def _zc(j, _):
            cntv[pl.ds(j * 16, 16)] = jnp.zeros((16,), F32)
            return 0
          lax.fori_loop(0, n_pad // 16, _zc, 0)

        plsc.subcore_barrier()

        ones = jnp.full((16,), 1.0, F32)

        pltpu.async_copy(table.at[sidx.at[pl.ds(0, 64)]], rows.at[0], sem)

        def _step(j, _):
          p = lax.rem(j, 2)
          pltpu.make_async_copy(table.at[sidx.at[pl.ds(0, 64)]],
                                rows.at[p], sem).wait()
          for q in range(4):
            dst_st[p, pl.ds(q * 16, 16)] = didx[pl.ds(j * 64 + q * 16, 16)]
          pltpu.async_copy(rows.at[p], acc.at[dst_st.at[p]], ssem, add=True)

          @pl.when(j > 0)
          def _wsc():
            pltpu.make_async_copy(rows.at[1 - p], acc.at[dst_st.at[1 - p]],
                                  ssem).wait()

          @pl.when(j < nch - 1)
          def _pref():
            pltpu.async_copy(table.at[sidx.at[pl.ds((j + 1) * 64, 64)]],
                             rows.at[1 - p], sem)

          if with_counts:
            for q in range(4):
              ii = didx[pl.ds(j * 64 + q * 16, 16)]
              plsc.addupdate_scatter(cntv, [ii], ones)
          return 0
        lax.fori_loop(0, nch, _step, 0)

        lp = (nch - 1) % 2
        pltpu.make_async_copy(rows.at[lp], acc.at[dst_st.at[lp]],
                              ssem).wait()

        plsc.subcore_barrier()

        pltpu.sync_copy(acc.at[pl.ds(base, rpt)], aout.at[pl.ds(base, rpt)])
        if with_counts:
          pltpu.sync_copy(cntv.at[pl.ds(0, n_pad)], cnts[ci].at[sid])

  outs_t = [None] * n_outs
  cnts_t = [None] * n_cnts
  for (core, ti, pi, row_base, nch, n_pad, oi, ci) in items:
    outs_t[oi] = jax.ShapeDtypeStruct((n_pad, H), F32)
    if with_counts:
      cnts_t[ci] = jax.ShapeDtypeStruct((16, n_pad), F32)
  out_type = outs_t + cnts_t

  scratch = [
      pltpu.VMEM((max_e,), jnp.int32),
      pltpu.VMEM((max_e,), jnp.int32),
      pltpu.VMEM((8, 64), jnp.int32),
      pltpu.VMEM((2, 64, H), F32),
      pltpu.VMEM_SHARED((_ACC_ROWS, H), F32),
      pltpu.SemaphoreType.DMA,
      pltpu.SemaphoreType.DMA,
  ]
  if with_counts:
    scratch.append(pltpu.VMEM((_ACC_ROWS,), F32))

  mesh = plsc.VectorSubcoreMesh(core_axis_name="c", subcore_axis_name="s")
  return pl.kernel(body, out_type=out_type, mesh=mesh,
                   scratch_types=scratch,
                   compiler_params=pltpu.CompilerParams(
                       needs_layout_passes=False))


def _cls_body(xp, xd, lsrc, ldst, out, pidx, didx, prow, drow, pacc):
  cid = lax.axis_index("c")
  sid = lax.axis_index("s")
  wid = sid * 2 + cid

  def _chunk(k, _):
    base = wid * 640 + k * 80
    pltpu.sync_copy(lsrc.at[pl.ds(base, 80)], pidx)
    pltpu.sync_copy(ldst.at[pl.ds(base, 80)], didx)
    pltpu.sync_copy(xp.at[pidx], prow)
    pltpu.sync_copy(xd.at[didx], drow)

    def _row(r, _):
      acc = jnp.zeros((16,), F32)
      for c in range(8):
        acc = acc + prow[r, pl.ds(c * 16, 16)] * drow[r, pl.ds(c * 16, 16)]
      pacc[k * 80 + r, :] = acc
      return 0
    lax.fori_loop(0, 80, _row, 0)
    return 0
  lax.fori_loop(0, 8, _chunk, 0)
  pltpu.sync_copy(pacc, out.at[pl.ds(wid * 640, 640)])


_cls_kernel = pl.kernel(
    _cls_body,
    out_type=jax.ShapeDtypeStruct((LP, 16), F32),
    mesh=plsc.VectorSubcoreMesh(core_axis_name="c", subcore_axis_name="s"),
    scratch_types=[
        pltpu.VMEM((80,), jnp.int32),
        pltpu.VMEM((80,), jnp.int32),
        pltpu.VMEM((80, H), F32),
        pltpu.VMEM((80, H), F32),
        pltpu.VMEM((640, 16), F32),
    ],
)


def _dense(n_rows, n_groups, agg_counts, act):
  BT = 1024
  n_pad = -(-n_rows // BT) * BT
  n_aggs = sum(agg_counts)

  def body(*refs):
    k = 0
    aggs = refs[k:k + n_aggs]; k += n_aggs
    cnts = refs[k:k + n_groups]; k += n_groups
    x_ref, wl_ref, wr_ref, b_ref, o_ref = refs[k:k + 5]

    wr_sum = jnp.sum(wr_ref[...], axis=0)
    acc = lax.dot_general(x_ref[...], wr_sum, (((1,), (1,)), ((), ())),
                          preferred_element_type=F32)
    acc = acc + jnp.sum(b_ref[...], axis=0)[None, :]
    a_at = 0
    for g in range(n_groups):
      a = aggs[a_at][...]
      for extra in range(1, agg_counts[g]):
        a = a + aggs[a_at + extra][...]
      a_at += agg_counts[g]
      c = jnp.sum(cnts[g][...], axis=0)
      mean = a * (1.0 / jnp.maximum(c, 1.0))[:, None]
      acc = acc + lax.dot_general(mean, wl_ref[g], (((1,), (1,)), ((), ())),
                                  preferred_element_type=F32)
    o_ref[...] = jnp.tanh(acc) if act else acc

  blk = pl.BlockSpec((BT, H), lambda i: (i, 0))
  in_specs = ([blk] * n_aggs
              + [pl.BlockSpec((16, BT), lambda i: (0, i))] * n_groups
              + [blk,
                 pl.BlockSpec((n_groups, H, H), lambda i: (0, 0, 0)),
                 pl.BlockSpec((n_groups, H, H), lambda i: (0, 0, 0)),
                 pl.BlockSpec((n_groups, H), lambda i: (0, 0))])
  return pl.pallas_call(
      body,
      grid=(n_pad // BT,),
      in_specs=in_specs,
      out_specs=blk,
      out_shape=jax.ShapeDtypeStruct((n_rows, H), F32),
  )


def _loss_body(part_ref, y_ref, pred_ref, loss_ref):
  pred = jnp.sum(part_ref[...], axis=1, keepdims=True)
  pred_ref[...] = pred
  y = y_ref[...]
  z = jnp.maximum(pred, 0.0) - pred * y + jnp.log1p(jnp.exp(-jnp.abs(pred)))
  loss_ref[...] = jnp.sum(z).reshape(1, 1) * (1.0 / L)


_loss_kernel = pl.pallas_call(
    _loss_body,
    out_shape=(jax.ShapeDtypeStruct((L, 1), F32),
               jax.ShapeDtypeStruct((1, 1), F32)),
)


_L1_ITEMS = []
for _i, (_st, _dt) in enumerate(_ET):
  _L1_ITEMS.append((_i % 2, _NT.index(_st), _i, 0, 160, _PAD[_dt], _i, _i))
_agg_l1 = _make_agg(_L1_ITEMS, with_counts=True)

_L2_ITEMS = [
    (0, _NT.index("symptom"), 0, 0, 160, _PAD["patient"], 0, 0),
    (0, _NT.index("procedure"), 1, 0, 160, _PAD["patient"], 1, 0),
    (1, _NT.index("disease"), 2, 0, 160, _PAD["patient"], 2, 0),
    (1, _NT.index("drug"), 3, 0, 160, _PAD["patient"], 3, 0),
    (0, _NT.index("patient"), 4, 0, 80, _PAD["drug"], 4, 0),
    (1, _NT.index("patient"), 4, 16, 80, _PAD["drug"], 5, 0),
]
_agg_l2 = _make_agg(_L2_ITEMS, with_counts=False)

_dense_p1 = _dense(_N["patient"], 4, (1, 1, 1, 1), act=True)
_dense_o1 = {t: _dense(_N[t], 1, (1,), act=True)
             for t in ["symptom", "procedure", "disease", "drug"]}
_dense_p2 = _dense(_N["patient"], 4, (1, 1, 1, 1), act=False)
_dense_d2 = _dense(_N["drug"], 1, (2,), act=False)


def kernel(node_id_patient, emb_patient, node_id_symptom, emb_symptom,
           node_id_procedure, emb_procedure, node_id_disease, emb_disease,
           node_id_drug, emb_drug,
           e0_src, e0_dst, e1_src, e1_dst, e2_src, e2_dst, e3_src, e3_dst,
           e4_src, e4_dst, e5_src, e5_dst, e6_src, e6_dst, e7_src, e7_dst,
           lbl_src, lbl_dst, edge_label,
           W1l, W1r, b1, W2l, W2r, b2):
  del node_id_patient, node_id_symptom, node_id_procedure, node_id_disease
  del node_id_drug
  tabs = {"patient": emb_patient, "symptom": emb_symptom,
          "procedure": emb_procedure, "disease": emb_disease,
          "drug": emb_drug}
  srcs = [e0_src, e1_src, e2_src, e3_src, e4_src, e5_src, e6_src, e7_src]
  dsts = [e0_dst, e1_dst, e2_dst, e3_dst, e4_dst, e5_dst, e6_dst, e7_dst]

  def _pad_edges(src, dst, n_dst, tiles, per):
    s2 = src.reshape(tiles, -1)
    padn = per - s2.shape[1]
    s2 = jnp.pad(s2, ((0, 0), (0, padn)))
    d2 = jnp.pad(dst.reshape(tiles, -1), ((0, 0), (0, padn)),
                 constant_values=n_dst)
    return s2.reshape(-1), d2.reshape(-1)

  args1 = [tabs[t] for t in _NT]
  for i in range(8):
    args1 += list(_pad_edges(srcs[i], dsts[i], _N[_ET[i][1]], 16, 10240))
  r1 = _agg_l1(*args1)
  aggs1, cnts1 = r1[:8], r1[8:]

  w1l = W1l.reshape(8, 1, H, H)
  w1r = W1r.reshape(8, 1, H, H)
  b1r = b1.reshape(8, 1, H)

  def sel(w, idxs):
    return jnp.concatenate([w[i] for i in idxs], axis=0)

  x1 = {}
  x1["patient"] = _dense_p1(
      aggs1[1], aggs1[3], aggs1[5], aggs1[7],
      cnts1[1], cnts1[3], cnts1[5], cnts1[7],
      tabs["patient"], sel(w1l, (1, 3, 5, 7)), sel(w1r, (1, 3, 5, 7)),
      sel(b1r, (1, 3, 5, 7)))
  for t, i in [("symptom", 0), ("procedure", 2), ("disease", 4), ("drug", 6)]:
    x1[t] = _dense_o1[t](aggs1[i], cnts1[i], tabs[t],
                         w1l[i], w1r[i], b1r[i])

  p1 = _pad_edges(srcs[1], dsts[1], _N["patient"], 16, 10240)
  p3 = _pad_edges(srcs[3], dsts[3], _N["patient"], 16, 10240)
  p5 = _pad_edges(srcs[5], dsts[5], _N["patient"], 16, 10240)
  p7 = _pad_edges(srcs[7], dsts[7], _N["patient"], 16, 10240)
  p6 = _pad_edges(srcs[6], dsts[6], _N["drug"], 32, 5120)
  a1, a3, a5, a7, a6a, a6b = _agg_l2(
      x1["patient"], x1["symptom"], x1["procedure"], x1["disease"],
      x1["drug"], *p1, *p3, *p5, *p7, *p6)

  w2l = W2l.reshape(8, 1, H, H)
  w2r = W2r.reshape(8, 1, H, H)
  b2r = b2.reshape(8, 1, H)
  x2p = _dense_p2(a1, a3, a5, a7,
                  cnts1[1], cnts1[3], cnts1[5], cnts1[7],
                  x1["patient"], sel(w2l, (1, 3, 5, 7)),
                  sel(w2r, (1, 3, 5, 7)), sel(b2r, (1, 3, 5, 7)))
  x2d = _dense_d2(a6a, a6b, cnts1[6], x1["drug"],
                  w2l[6], w2r[6], b2r[6])

  ls = jnp.pad(lbl_src, (0, LP - L))
  ld = jnp.pad(lbl_dst, (0, LP - L))
  part = _cls_kernel(x2p, x2d, ls, ld)

  y = edge_label.astype(F32).reshape(L, 1)
  pred2d, loss2d = _loss_kernel(part[:L], y)
  return (loss2d[0, 0], pred2d[:, 0])


# speedup vs baseline: 1.1736x; 1.1736x over previous
import jax
import jax.numpy as jnp
from jax import lax
from jax.experimental import pallas as pl
from jax.experimental.pallas import tpu as pltpu
from jax.experimental.pallas import tpu_sc as plsc

F32 = jnp.float32
H = 128
E = 160000
L = 20000
LP = 20480

_NT = ["patient", "symptom", "procedure", "disease", "drug"]
_N = {"patient": 10000, "symptom": 5000, "procedure": 5000, "disease": 3000,
      "drug": 2000}
_PAD = {"patient": 10240, "symptom": 5120, "procedure": 5120, "disease": 3072,
        "drug": 2048}
_ET = [("patient", "symptom"), ("symptom", "patient"),
       ("patient", "procedure"), ("procedure", "patient"),
       ("patient", "disease"), ("disease", "patient"),
       ("patient", "drug"), ("drug", "patient")]

_ACC_ROWS = 10240


def _make_agg(items, with_counts):
  n_tables = 1 + max(it[1] for it in items)
  n_pairs = 1 + max(it[2] for it in items)
  n_outs = 1 + max(it[6] for it in items)
  n_cnts = (1 + max(it[7] for it in items)) if with_counts else 0
  max_e = 64 * max(it[4] for it in items)

  def body(*refs):
    k = 0
    tables = refs[k:k + n_tables]; k += n_tables
    pairs = [(refs[k + 2 * i], refs[k + 2 * i + 1]) for i in range(n_pairs)]
    k += 2 * n_pairs
    outs = refs[k:k + n_outs]; k += n_outs
    cnts = refs[k:k + n_cnts]; k += n_cnts
    sst, dst_st, rows, acc, isem, gsem, ssem = refs[k:k + 7]
    k += 7
    cntv = refs[k] if with_counts else None

    cid = lax.axis_index("c")
    sid = lax.axis_index("s")

    for (core, ti, pi, row_base, nch, n_pad, oi, ci) in items:

      @pl.when(cid == core)
      def _item(table=tables[ti], src=pairs[pi][0], dst=pairs[pi][1],
                aout=outs[oi], ci=ci, row_base=row_base, nch=nch,
                n_pad=n_pad):
        rpt = n_pad // 16
        base = sid * rpt
        ebase = (row_base + sid) * 64 * nch

        def _issue_idx(c):
          s = lax.rem(c, 8)
          pltpu.async_copy(src.at[pl.ds(ebase + c * 64, 64)], sst.at[s],
                           isem)
          pltpu.async_copy(dst.at[pl.ds(ebase + c * 64, 64)], dst_st.at[s],
                           isem)

        def _wait_idx():
          pltpu.make_async_copy(src.at[pl.ds(0, 64)], sst.at[0],
                                isem).wait()
          pltpu.make_async_copy(dst.at[pl.ds(0, 64)], dst_st.at[0],
                                isem).wait()

        def _issue_gather(c):
          pltpu.async_copy(table.at[sst.at[lax.rem(c, 8)]],
                           rows.at[lax.rem(c, 4)], gsem)

        for c in range(4):
          _issue_idx(c)

        def _zb(i, _):
          rows[0, i // 8, pl.ds((i % 8) * 16, 16)] = jnp.zeros((16,), F32)
          return 0
        lax.fori_loop(0, 512, _zb, 0)

        def _zr(j, _):
          pltpu.async_copy(rows.at[0], acc.at[pl.ds(base + j * 64, 64)], gsem)
          return 0
        lax.fori_loop(0, rpt // 64, _zr, 0)

        def _zw(j, _):
          pltpu.make_async_copy(rows.at[0], acc.at[pl.ds(base, 64)],
                                gsem).wait()
          return 0
        lax.fori_loop(0, rpt // 64, _zw, 0)

        if with_counts:
          def _zc(j, _):
            cntv[pl.ds(j * 16, 16)] = jnp.zeros((16,), F32)
            return 0
          lax.fori_loop(0, n_pad // 16, _zc, 0)

        plsc.subcore_barrier()

        ones = jnp.full((16,), 1.0, F32)

        def _wait_gather(c):
          pltpu.make_async_copy(table.at[sst.at[0]], rows.at[lax.rem(c, 4)],
                                gsem).wait()

        def _wait_scatter(c):
          pltpu.make_async_copy(rows.at[0], acc.at[dst_st.at[0]],
                                ssem).wait()

        _wait_idx()
        _issue_gather(0)
        _wait_idx()
        _issue_gather(1)

        def _step(j, _):
          _wait_gather(j)
          pltpu.async_copy(rows.at[lax.rem(j, 4)],
                           acc.at[dst_st.at[lax.rem(j, 8)]], ssem, add=True)
          if with_counts:
            for q in range(4):
              ii = dst_st[lax.rem(j, 8), pl.ds(q * 16, 16)]
              plsc.addupdate_scatter(cntv, [ii], ones)

          @pl.when(j >= 2)
          def _wsc():
            _wait_scatter(j - 2)

          @pl.when(j + 4 < nch)
          def _pidx():
            _issue_idx(j + 4)

          @pl.when(j + 2 < nch)
          def _pg():
            _wait_idx()
            _issue_gather(j + 2)
          return 0
        lax.fori_loop(0, nch, _step, 0)

        _wait_scatter(nch - 2)
        _wait_scatter(nch - 1)

        plsc.subcore_barrier()

        pltpu.sync_copy(acc.at[pl.ds(base, rpt)], aout.at[pl.ds(base, rpt)])
        if with_counts:
          pltpu.sync_copy(cntv.at[pl.ds(0, n_pad)], cnts[ci].at[sid])

  outs_t = [None] * n_outs
  cnts_t = [None] * n_cnts
  for (core, ti, pi, row_base, nch, n_pad, oi, ci) in items:
    outs_t[oi] = jax.ShapeDtypeStruct((n_pad, H), F32)
    if with_counts:
      cnts_t[ci] = jax.ShapeDtypeStruct((16, n_pad), F32)
  out_type = outs_t + cnts_t

  scratch = [
      pltpu.VMEM((8, 64), jnp.int32),
      pltpu.VMEM((8, 64), jnp.int32),
      pltpu.VMEM((4, 64, H), F32),
      pltpu.VMEM_SHARED((_ACC_ROWS, H), F32),
      pltpu.SemaphoreType.DMA,
      pltpu.SemaphoreType.DMA,
      pltpu.SemaphoreType.DMA,
  ]
  if with_counts:
    scratch.append(pltpu.VMEM((_ACC_ROWS,), F32))

  mesh = plsc.VectorSubcoreMesh(core_axis_name="c", subcore_axis_name="s")
  return pl.kernel(body, out_type=out_type, mesh=mesh,
                   scratch_types=scratch,
                   compiler_params=pltpu.CompilerParams(
                       needs_layout_passes=False))


def _cls_body(xp, xd, lsrc, ldst, out, pidx, didx, prow, drow, pacc):
  cid = lax.axis_index("c")
  sid = lax.axis_index("s")
  wid = sid * 2 + cid

  def _chunk(k, _):
    base = wid * 640 + k * 80
    pltpu.sync_copy(lsrc.at[pl.ds(base, 80)], pidx)
    pltpu.sync_copy(ldst.at[pl.ds(base, 80)], didx)
    pltpu.sync_copy(xp.at[pidx], prow)
    pltpu.sync_copy(xd.at[didx], drow)

    def _row(r, _):
      acc = jnp.zeros((16,), F32)
      for c in range(8):
        acc = acc + prow[r, pl.ds(c * 16, 16)] * drow[r, pl.ds(c * 16, 16)]
      pacc[k * 80 + r, :] = acc
      return 0
    lax.fori_loop(0, 80, _row, 0)
    return 0
  lax.fori_loop(0, 8, _chunk, 0)
  pltpu.sync_copy(pacc, out.at[pl.ds(wid * 640, 640)])


_cls_kernel = pl.kernel(
    _cls_body,
    out_type=jax.ShapeDtypeStruct((LP, 16), F32),
    mesh=plsc.VectorSubcoreMesh(core_axis_name="c", subcore_axis_name="s"),
    scratch_types=[
        pltpu.VMEM((80,), jnp.int32),
        pltpu.VMEM((80,), jnp.int32),
        pltpu.VMEM((80, H), F32),
        pltpu.VMEM((80, H), F32),
        pltpu.VMEM((640, 16), F32),
    ],
)


def _dense(n_rows, n_groups, agg_counts, act):
  BT = 1024
  n_pad = -(-n_rows // BT) * BT
  n_aggs = sum(agg_counts)

  def body(*refs):
    k = 0
    aggs = refs[k:k + n_aggs]; k += n_aggs
    cnts = refs[k:k + n_groups]; k += n_groups
    x_ref, wl_ref, wr_ref, b_ref, o_ref = refs[k:k + 5]

    wr_sum = jnp.sum(wr_ref[...], axis=0)
    acc = lax.dot_general(x_ref[...], wr_sum, (((1,), (1,)), ((), ())),
                          preferred_element_type=F32)
    acc = acc + jnp.sum(b_ref[...], axis=0)[None, :]
    a_at = 0
    for g in range(n_groups):
      a = aggs[a_at][...]
      for extra in range(1, agg_counts[g]):
        a = a + aggs[a_at + extra][...]
      a_at += agg_counts[g]
      c = jnp.sum(cnts[g][...], axis=0)
      mean = a * (1.0 / jnp.maximum(c, 1.0))[:, None]
      acc = acc + lax.dot_general(mean, wl_ref[g], (((1,), (1,)), ((), ())),
                                  preferred_element_type=F32)
    o_ref[...] = jnp.tanh(acc) if act else acc

  blk = pl.BlockSpec((BT, H), lambda i: (i, 0))
  in_specs = ([blk] * n_aggs
              + [pl.BlockSpec((16, BT), lambda i: (0, i))] * n_groups
              + [blk,
                 pl.BlockSpec((n_groups, H, H), lambda i: (0, 0, 0)),
                 pl.BlockSpec((n_groups, H, H), lambda i: (0, 0, 0)),
                 pl.BlockSpec((n_groups, H), lambda i: (0, 0))])
  return pl.pallas_call(
      body,
      grid=(n_pad // BT,),
      in_specs=in_specs,
      out_specs=blk,
      out_shape=jax.ShapeDtypeStruct((n_rows, H), F32),
  )


def _loss_body(part_ref, y_ref, pred_ref, loss_ref):
  pred = jnp.sum(part_ref[...], axis=1, keepdims=True)
  pred_ref[...] = pred
  y = y_ref[...]
  z = jnp.maximum(pred, 0.0) - pred * y + jnp.log1p(jnp.exp(-jnp.abs(pred)))
  loss_ref[...] = jnp.sum(z).reshape(1, 1) * (1.0 / L)


_loss_kernel = pl.pallas_call(
    _loss_body,
    out_shape=(jax.ShapeDtypeStruct((L, 1), F32),
               jax.ShapeDtypeStruct((1, 1), F32)),
)


_L1_ITEMS = []
for _i, (_st, _dt) in enumerate(_ET):
  _L1_ITEMS.append((_i % 2, _NT.index(_st), _i, 0, 160, _PAD[_dt], _i, _i))
_agg_l1 = _make_agg(_L1_ITEMS, with_counts=True)

_L2_ITEMS = [
    (0, _NT.index("symptom"), 0, 0, 160, _PAD["patient"], 0, 0),
    (0, _NT.index("procedure"), 1, 0, 160, _PAD["patient"], 1, 0),
    (1, _NT.index("disease"), 2, 0, 160, _PAD["patient"], 2, 0),
    (1, _NT.index("drug"), 3, 0, 160, _PAD["patient"], 3, 0),
    (0, _NT.index("patient"), 4, 0, 80, _PAD["drug"], 4, 0),
    (1, _NT.index("patient"), 4, 16, 80, _PAD["drug"], 5, 0),
]
_agg_l2 = _make_agg(_L2_ITEMS, with_counts=False)

_dense_p1 = _dense(_N["patient"], 4, (1, 1, 1, 1), act=True)
_dense_o1 = {t: _dense(_N[t], 1, (1,), act=True)
             for t in ["symptom", "procedure", "disease", "drug"]}
_dense_p2 = _dense(_N["patient"], 4, (1, 1, 1, 1), act=False)
_dense_d2 = _dense(_N["drug"], 1, (2,), act=False)


def kernel(node_id_patient, emb_patient, node_id_symptom, emb_symptom,
           node_id_procedure, emb_procedure, node_id_disease, emb_disease,
           node_id_drug, emb_drug,
           e0_src, e0_dst, e1_src, e1_dst, e2_src, e2_dst, e3_src, e3_dst,
           e4_src, e4_dst, e5_src, e5_dst, e6_src, e6_dst, e7_src, e7_dst,
           lbl_src, lbl_dst, edge_label,
           W1l, W1r, b1, W2l, W2r, b2):
  del node_id_patient, node_id_symptom, node_id_procedure, node_id_disease
  del node_id_drug
  tabs = {"patient": emb_patient, "symptom": emb_symptom,
          "procedure": emb_procedure, "disease": emb_disease,
          "drug": emb_drug}
  srcs = [e0_src, e1_src, e2_src, e3_src, e4_src, e5_src, e6_src, e7_src]
  dsts = [e0_dst, e1_dst, e2_dst, e3_dst, e4_dst, e5_dst, e6_dst, e7_dst]

  def _pad_edges(src, dst, n_dst, tiles, per):
    s2 = src.reshape(tiles, -1)
    padn = per - s2.shape[1]
    s2 = jnp.pad(s2, ((0, 0), (0, padn)))
    d2 = jnp.pad(dst.reshape(tiles, -1), ((0, 0), (0, padn)),
                 constant_values=n_dst)
    return s2.reshape(-1), d2.reshape(-1)

  args1 = [tabs[t] for t in _NT]
  for i in range(8):
    args1 += list(_pad_edges(srcs[i], dsts[i], _N[_ET[i][1]], 16, 10240))
  r1 = _agg_l1(*args1)
  aggs1, cnts1 = r1[:8], r1[8:]

  w1l = W1l.reshape(8, 1, H, H)
  w1r = W1r.reshape(8, 1, H, H)
  b1r = b1.reshape(8, 1, H)

  def sel(w, idxs):
    return jnp.concatenate([w[i] for i in idxs], axis=0)

  x1 = {}
  x1["patient"] = _dense_p1(
      aggs1[1], aggs1[3], aggs1[5], aggs1[7],
      cnts1[1], cnts1[3], cnts1[5], cnts1[7],
      tabs["patient"], sel(w1l, (1, 3, 5, 7)), sel(w1r, (1, 3, 5, 7)),
      sel(b1r, (1, 3, 5, 7)))
  for t, i in [("symptom", 0), ("procedure", 2), ("disease", 4), ("drug", 6)]:
    x1[t] = _dense_o1[t](aggs1[i], cnts1[i], tabs[t],
                         w1l[i], w1r[i], b1r[i])

  p1 = _pad_edges(srcs[1], dsts[1], _N["patient"], 16, 10240)
  p3 = _pad_edges(srcs[3], dsts[3], _N["patient"], 16, 10240)
  p5 = _pad_edges(srcs[5], dsts[5], _N["patient"], 16, 10240)
  p7 = _pad_edges(srcs[7], dsts[7], _N["patient"], 16, 10240)
  p6 = _pad_edges(srcs[6], dsts[6], _N["drug"], 32, 5120)
  a1, a3, a5, a7, a6a, a6b = _agg_l2(
      x1["patient"], x1["symptom"], x1["procedure"], x1["disease"],
      x1["drug"], *p1, *p3, *p5, *p7, *p6)

  w2l = W2l.reshape(8, 1, H, H)
  w2r = W2r.reshape(8, 1, H, H)
  b2r = b2.reshape(8, 1, H)
  x2p = _dense_p2(a1, a3, a5, a7,
                  cnts1[1], cnts1[3], cnts1[5], cnts1[7],
                  x1["patient"], sel(w2l, (1, 3, 5, 7)),
                  sel(w2r, (1, 3, 5, 7)), sel(b2r, (1, 3, 5, 7)))
  x2d = _dense_d2(a6a, a6b, cnts1[6], x1["drug"],
                  w2l[6], w2r[6], b2r[6])

  ls = jnp.pad(lbl_src, (0, LP - L))
  ld = jnp.pad(lbl_dst, (0, LP - L))
  part = _cls_kernel(x2p, x2d, ls, ld)

  y = edge_label.astype(F32).reshape(L, 1)
  pred2d, loss2d = _loss_kernel(part[:L], y)
  return (loss2d[0, 0], pred2d[:, 0])
